# Initial kernel scaffold; baseline (speedup 1.0000x reference)
import jax
import jax.numpy as jnp
from jax.experimental import pallas as pl


def kernel(x, W1, b1, W2, b2):
    raise NotImplementedError("write your pallas kernel here")


# speedup vs baseline: 1.4653x; 1.4653x over previous
import jax
import jax.numpy as jnp
from jax import lax
from jax.experimental import pallas as pl
from jax.experimental.pallas import tpu as pltpu

_B, _N, _D, _H = 8, 65536, 7, 64
_GH, _GW = 32, 32
_S = _GH * _GW
_C = 2048
_NC = _N // _C


def _fused_kernel(x_ref, W1_ref, b1_ref, W2_ref, b2_ref, out_ref, spans):
    phase = pl.program_id(1)
    i = pl.program_id(2)
    xb = x_ref[0]
    c0 = xb[:, 0:1]
    c1 = xb[:, 1:2]

    @pl.when(phase == 0)
    def _minmax():
        @pl.when(i == 0)
        def _init():
            spans[0] = jnp.float32(jnp.inf)
            spans[1] = jnp.float32(-jnp.inf)
            spans[2] = jnp.float32(jnp.inf)
            spans[3] = jnp.float32(-jnp.inf)

        spans[0] = jnp.minimum(spans[0], jnp.min(c0))
        spans[1] = jnp.maximum(spans[1], jnp.max(c0))
        spans[2] = jnp.minimum(spans[2], jnp.min(c1))
        spans[3] = jnp.maximum(spans[3], jnp.max(c1))

    @pl.when(phase == 1)
    def _encode():
        x_min = spans[0]
        x_span = jnp.maximum(spans[1] - spans[0], 1e-8)
        y_min = spans[2]
        y_span = jnp.maximum(spans[3] - spans[2], 1e-8)
        gx = jnp.clip(((c0 - x_min) / x_span * _GH).astype(jnp.int32), 0, _GH - 1)
        gy = jnp.clip(((c1 - y_min) / y_span * _GW).astype(jnp.int32), 0, _GW - 1)
        seg = gx * _GW + gy

        h = jnp.maximum(
            jnp.dot(xb, W1_ref[...], preferred_element_type=jnp.float32)
            + b1_ref[...],
            0.0,
        )
        phi = (
            jnp.dot(h, W2_ref[...], preferred_element_type=jnp.float32)
            + b2_ref[...]
        )

        onehot = (
            seg == lax.broadcasted_iota(jnp.int32, (_C, _S), 1)
        ).astype(jnp.float32)
        part = lax.dot_general(
            onehot,
            phi,
            (((0,), (0,)), ((), ())),
            preferred_element_type=jnp.float32,
        )

        @pl.when(i == 0)
        def _store():
            out_ref[0] = part

        @pl.when(i > 0)
        def _acc():
            out_ref[0] += part


def kernel(x, W1, b1, W2, b2):
    out = pl.pallas_call(
        _fused_kernel,
        grid=(_B, 2, _NC),
        in_specs=[
            pl.BlockSpec((1, _C, _D), lambda b, p, i: (b, i, 0)),
            pl.BlockSpec((_D, _H), lambda b, p, i: (0, 0)),
            pl.BlockSpec((1, _H), lambda b, p, i: (0, 0)),
            pl.BlockSpec((_H, _H), lambda b, p, i: (0, 0)),
            pl.BlockSpec((1, _H), lambda b, p, i: (0, 0)),
        ],
        out_specs=pl.BlockSpec((1, _S, _H), lambda b, p, i: (b, 0, 0)),
        out_shape=jax.ShapeDtypeStruct((_B, _S, _H), jnp.float32),
        scratch_shapes=[pltpu.SMEM((4,), jnp.float32)],
    )(x, W1, b1.reshape(1, _H), W2, b2.reshape(1, _H))
    return out.reshape(_B, _GH, _GW, _H)


# speedup vs baseline: 1.6720x; 1.1410x over previous
import jax
import jax.numpy as jnp
from jax import lax
from jax.experimental import pallas as pl
from jax.experimental.pallas import tpu as pltpu

_B, _N, _D, _H = 8, 65536, 7, 64
_GH, _GW = 32, 32
_S = _GH * _GW
_C = 2048
_NC = _N // _C


def _fused_kernel(x_ref, W1_ref, b1_ref, W2_ref, b2_ref, out_ref, spans):
    phase = pl.program_id(1)
    i = pl.program_id(2)
    xb = x_ref[0]
    c0 = xb[:, 0:1]
    c1 = xb[:, 1:2]

    @pl.when(phase == 0)
    def _minmax():
        @pl.when(i == 0)
        def _init():
            spans[0] = jnp.float32(jnp.inf)
            spans[1] = jnp.float32(-jnp.inf)
            spans[2] = jnp.float32(jnp.inf)
            spans[3] = jnp.float32(-jnp.inf)

        spans[0] = jnp.minimum(spans[0], jnp.min(c0))
        spans[1] = jnp.maximum(spans[1], jnp.max(c0))
        spans[2] = jnp.minimum(spans[2], jnp.min(c1))
        spans[3] = jnp.maximum(spans[3], jnp.max(c1))

    @pl.when(phase == 1)
    def _encode():
        x_min = spans[0]
        x_span = jnp.maximum(spans[1] - spans[0], 1e-8)
        y_min = spans[2]
        y_span = jnp.maximum(spans[3] - spans[2], 1e-8)
        gx = jnp.clip(((c0 - x_min) / x_span * _GH).astype(jnp.int32), 0, _GH - 1)
        gy = jnp.clip(((c1 - y_min) / y_span * _GW).astype(jnp.int32), 0, _GW - 1)
        seg = gx * _GW + gy

        h = jnp.maximum(
            jnp.dot(xb, W1_ref[...], preferred_element_type=jnp.float32)
            + b1_ref[...],
            0.0,
        )
        phi = (
            jnp.dot(h, W2_ref[...], preferred_element_type=jnp.float32)
            + b2_ref[...]
        )

        onehot = (
            seg == lax.broadcasted_iota(jnp.int32, (_C, _S), 1)
        ).astype(jnp.bfloat16)
        part = lax.dot_general(
            onehot,
            phi.astype(jnp.bfloat16),
            (((0,), (0,)), ((), ())),
            preferred_element_type=jnp.float32,
        )

        @pl.when(i == 0)
        def _store():
            out_ref[0] = part

        @pl.when(i > 0)
        def _acc():
            out_ref[0] += part


def kernel(x, W1, b1, W2, b2):
    out = pl.pallas_call(
        _fused_kernel,
        grid=(_B, 2, _NC),
        in_specs=[
            pl.BlockSpec((1, _C, _D), lambda b, p, i: (b, i, 0)),
            pl.BlockSpec((_D, _H), lambda b, p, i: (0, 0)),
            pl.BlockSpec((1, _H), lambda b, p, i: (0, 0)),
            pl.BlockSpec((_H, _H), lambda b, p, i: (0, 0)),
            pl.BlockSpec((1, _H), lambda b, p, i: (0, 0)),
        ],
        out_specs=pl.BlockSpec((1, _S, _H), lambda b, p, i: (b, 0, 0)),
        out_shape=jax.ShapeDtypeStruct((_B, _S, _H), jnp.float32),
        scratch_shapes=[pltpu.SMEM((4,), jnp.float32)],
    )(x, W1, b1.reshape(1, _H), W2, b2.reshape(1, _H))
    return out.reshape(_B, _GH, _GW, _H)


# speedup vs baseline: 1.9569x; 1.1704x over previous
import jax
import jax.numpy as jnp
from jax import lax
from jax.experimental import pallas as pl
from jax.experimental.pallas import tpu as pltpu

_B, _N, _D, _H = 8, 65536, 7, 64
_GH, _GW = 32, 32
_S = _GH * _GW
_C = 4096
_NC = _N // _C


def _fused_kernel(x_ref, W1_ref, b1_ref, W2_ref, b2_ref, out_ref, spans):
    phase = pl.program_id(1)
    i = pl.program_id(2)

    @pl.when(phase == 0)
    def _minmax():
        xb = x_ref[0]
        colmin = jnp.min(xb, axis=0, keepdims=True)
        colmax = jnp.max(xb, axis=0, keepdims=True)

        @pl.when(i == 0)
        def _init():
            spans[0] = jnp.float32(jnp.inf)
            spans[1] = jnp.float32(-jnp.inf)
            spans[2] = jnp.float32(jnp.inf)
            spans[3] = jnp.float32(-jnp.inf)

        spans[0] = jnp.minimum(spans[0], colmin[0, 0])
        spans[1] = jnp.maximum(spans[1], colmax[0, 0])
        spans[2] = jnp.minimum(spans[2], colmin[0, 1])
        spans[3] = jnp.maximum(spans[3], colmax[0, 1])

    @pl.when(phase == 1)
    def _encode():
        xb = x_ref[0]
        c0 = xb[:, 0:1]
        c1 = xb[:, 1:2]
        x_min = spans[0]
        x_span = jnp.maximum(spans[1] - spans[0], 1e-8)
        y_min = spans[2]
        y_span = jnp.maximum(spans[3] - spans[2], 1e-8)
        gx = jnp.clip(((c0 - x_min) / x_span * _GH).astype(jnp.int32), 0, _GH - 1)
        gy = jnp.clip(((c1 - y_min) / y_span * _GW).astype(jnp.int32), 0, _GW - 1)
        seg = gx * _GW + gy

        h = jnp.maximum(
            jnp.dot(xb, W1_ref[...], preferred_element_type=jnp.float32)
            + b1_ref[...],
            0.0,
        )
        phi = (
            jnp.dot(h, W2_ref[...], preferred_element_type=jnp.float32)
            + b2_ref[...]
        )

        onehot = (
            seg == lax.broadcasted_iota(jnp.int32, (_C, _S), 1)
        ).astype(jnp.bfloat16)
        part = lax.dot_general(
            onehot,
            phi.astype(jnp.bfloat16),
            (((0,), (0,)), ((), ())),
            preferred_element_type=jnp.float32,
        )

        @pl.when(i == 0)
        def _store():
            out_ref[0] = part

        @pl.when(i > 0)
        def _acc():
            out_ref[0] += part


def kernel(x, W1, b1, W2, b2):
    out = pl.pallas_call(
        _fused_kernel,
        grid=(_B, 2, _NC),
        in_specs=[
            pl.BlockSpec((1, _C, _D), lambda b, p, i: (b, i, 0)),
            pl.BlockSpec((_D, _H), lambda b, p, i: (0, 0)),
            pl.BlockSpec((1, _H), lambda b, p, i: (0, 0)),
            pl.BlockSpec((_H, _H), lambda b, p, i: (0, 0)),
            pl.BlockSpec((1, _H), lambda b, p, i: (0, 0)),
        ],
        out_specs=pl.BlockSpec((1, _S, _H), lambda b, p, i: (b, 0, 0)),
        out_shape=jax.ShapeDtypeStruct((_B, _S, _H), jnp.float32),
        scratch_shapes=[pltpu.SMEM((4,), jnp.float32)],
    )(x, W1, b1.reshape(1, _H), W2, b2.reshape(1, _H))
    return out.reshape(_B, _GH, _GW, _H)
